# baseline (speedup 1.0000x reference)
import functools

import jax
import jax.numpy as jnp
from jax import lax
from jax.experimental import pallas as pl
from jax.experimental.pallas import tpu as pltpu
from jax.experimental.pallas import tpu_sc as plsc

_NC = 2
_NS = 16
_NW = _NC * _NS


def _build_gather(B: int, V: int, E: int, C: int):
    assert B % (_NW * C) == 0
    b_per_w = B // _NW
    nchunks = b_per_w // C
    assert nchunks >= 4 and nchunks % 2 == 0

    mesh = plsc.VectorSubcoreMesh(core_axis_name="c", subcore_axis_name="s")

    @functools.partial(
        pl.kernel,
        out_type=jax.ShapeDtypeStruct((B, E), jnp.float32),
        mesh=mesh,
        compiler_params=pltpu.CompilerParams(use_tc_tiling_on_sc=False),
        scratch_types=[
            pltpu.VMEM((C,), jnp.int32),
            pltpu.VMEM((C,), jnp.int32),
            pltpu.VMEM((C, E), jnp.float32),
            pltpu.VMEM((C, E), jnp.float32),
            pltpu.SemaphoreType.DMA,
            pltpu.SemaphoreType.DMA,
            pltpu.SemaphoreType.DMA,
            pltpu.SemaphoreType.DMA,
            pltpu.SemaphoreType.DMA,
            pltpu.SemaphoreType.DMA,
        ],
    )
    def gather(idx_hbm, table_hbm, out_hbm, idx_v0, idx_v1, rows_v0, rows_v1, *sems):
        idx_v = (idx_v0, idx_v1)
        rows_v = (rows_v0, rows_v1)
        sem_i = sems[0:2]
        sem_g = sems[2:4]
        sem_o = sems[4:6]
        wid = lax.axis_index("s") * _NC + lax.axis_index("c")
        base = wid * b_per_w

        def start_idx(g, b):
            pltpu.make_async_copy(
                idx_hbm.at[pl.ds(base + g * C, C)], idx_v[b], sem_i[b]
            ).start()

        def wait_idx(b):
            pltpu.make_async_copy(
                idx_hbm.at[pl.ds(base, C)], idx_v[b], sem_i[b]
            ).wait()

        def start_gather(b):
            pltpu.make_async_copy(
                table_hbm.at[idx_v[b]], rows_v[b], sem_g[b]
            ).start()

        def wait_gather(b):
            pltpu.make_async_copy(
                table_hbm.at[idx_v[b]], rows_v[b], sem_g[b]
            ).wait()

        def start_out(g, b):
            pltpu.make_async_copy(
                rows_v[b], out_hbm.at[pl.ds(base + g * C, C)], sem_o[b]
            ).start()

        def wait_out(b):
            pltpu.make_async_copy(
                rows_v[b], out_hbm.at[pl.ds(base, C)], sem_o[b]
            ).wait()

        def step(g, b, first=False, last=False):
            nb = 1 - b
            if not first:
                wait_out(nb)
            wait_idx(nb)
            start_gather(nb)
            wait_gather(b)
            if not last:
                start_idx(g + 2, b)
            start_out(g, b)

        start_idx(0, 0)
        start_idx(1, 1)
        wait_idx(0)
        start_gather(0)
        step(0, 0, first=True)
        step(1, 1)

        def loop_body(i, _):
            g = 2 * i
            step(g, 0)
            step(g + 1, 1)
            return _

        lax.fori_loop(1, nchunks // 2 - 1, loop_body, 0, unroll=False)

        step(nchunks - 2, 0, last=True)
        wait_gather(1)
        start_out(nchunks - 1, 1)
        wait_out(0)
        wait_out(1)

    return gather


def kernel(p, table):
    N, D = p.shape
    V, E = table.shape
    K = 4
    Nk = N // K
    g = _build_gather(Nk * D, V, E, C=800)
    out = jnp.empty((N, D * E), jnp.float32)
    for i in range(K):
        flat = g(p[i * Nk:(i + 1) * Nk].reshape(Nk * D), table)
        out = lax.dynamic_update_slice(out, flat.reshape(Nk, D * E), (i * Nk, 0))
    return out


# speedup vs baseline: 1.0665x; 1.0665x over previous
import functools

import jax
import jax.numpy as jnp
from jax import lax
from jax.experimental import pallas as pl
from jax.experimental.pallas import tpu as pltpu
from jax.experimental.pallas import tpu_sc as plsc

_NC = 2
_NS = 16
_NW = _NC * _NS


def _build_gather(B: int, V: int, E: int, C: int):
    assert B % (_NW * C) == 0
    b_per_w = B // _NW
    nchunks = b_per_w // C
    assert nchunks >= 4 and nchunks % 2 == 0

    mesh = plsc.VectorSubcoreMesh(core_axis_name="c", subcore_axis_name="s")

    @functools.partial(
        pl.kernel,
        out_type=jax.ShapeDtypeStruct((B, E), jnp.float32),
        mesh=mesh,
        compiler_params=pltpu.CompilerParams(use_tc_tiling_on_sc=False),
        scratch_types=[
            pltpu.VMEM((C,), jnp.int32),
            pltpu.VMEM((C,), jnp.int32),
            pltpu.VMEM((C, E), jnp.float32),
            pltpu.VMEM((C, E), jnp.float32),
            pltpu.SemaphoreType.DMA,
            pltpu.SemaphoreType.DMA,
            pltpu.SemaphoreType.DMA,
            pltpu.SemaphoreType.DMA,
            pltpu.SemaphoreType.DMA,
            pltpu.SemaphoreType.DMA,
        ],
    )
    def gather(idx_hbm, table_hbm, out_hbm, idx_v0, idx_v1, rows_v0, rows_v1, *sems):
        idx_v = (idx_v0, idx_v1)
        rows_v = (rows_v0, rows_v1)
        sem_i = sems[0:2]
        sem_g = sems[2:4]
        sem_o = sems[4:6]
        wid = lax.axis_index("s") * _NC + lax.axis_index("c")
        base = wid * b_per_w

        def start_idx(g, b):
            pltpu.make_async_copy(
                idx_hbm.at[pl.ds(base + g * C, C)], idx_v[b], sem_i[b]
            ).start()

        def wait_idx(b):
            pltpu.make_async_copy(
                idx_hbm.at[pl.ds(base, C)], idx_v[b], sem_i[b]
            ).wait()

        def start_gather(b):
            pltpu.make_async_copy(
                table_hbm.at[idx_v[b]], rows_v[b], sem_g[b]
            ).start()

        def wait_gather(b):
            pltpu.make_async_copy(
                table_hbm.at[idx_v[b]], rows_v[b], sem_g[b]
            ).wait()

        def start_out(g, b):
            pltpu.make_async_copy(
                rows_v[b], out_hbm.at[pl.ds(base + g * C, C)], sem_o[b]
            ).start()

        def wait_out(b):
            pltpu.make_async_copy(
                rows_v[b], out_hbm.at[pl.ds(base, C)], sem_o[b]
            ).wait()

        def step(g, b, first=False, last=False):
            nb = 1 - b
            if not first:
                wait_out(nb)
            wait_idx(nb)
            start_gather(nb)
            wait_gather(b)
            if not last:
                start_idx(g + 2, b)
            start_out(g, b)

        start_idx(0, 0)
        start_idx(1, 1)
        wait_idx(0)
        start_gather(0)
        step(0, 0, first=True)
        step(1, 1)

        def loop_body(i, _):
            g = 2 * i
            step(g, 0)
            step(g + 1, 1)
            return _

        lax.fori_loop(1, nchunks // 2 - 1, loop_body, 0, unroll=False)

        step(nchunks - 2, 0, last=True)
        wait_gather(1)
        start_out(nchunks - 1, 1)
        wait_out(0)
        wait_out(1)

    return gather


def kernel(p, table):
    N, D = p.shape
    V, E = table.shape
    K = 2
    Nk = N // K
    g = _build_gather(Nk * D, V, E, C=800)
    parts = []
    for i in range(K):
        flat = g(p[i * Nk:(i + 1) * Nk].reshape(Nk * D), table)
        parts.append(flat.reshape(Nk, D * E))
    return jnp.concatenate(parts, axis=0)
